# baseline (speedup 1.0000x reference)
import jax
import jax.numpy as jnp
from jax import lax
from jax.experimental import pallas as pl
from jax.experimental.pallas import tpu as pltpu
from jax.experimental.pallas import tpu_sc as plsc

NUM_PERM = 8
C = 8
N = 1024
D = 768

_ROWS = NUM_PERM * C * N
_NC = 2
_NS = 16
_NW = _NC * _NS
_TEC_ROWS = _ROWS // _NW
_CH_ROWS = 64
_NCH = _TEC_ROWS // _CH_ROWS
_ACHUNK = (NUM_PERM * N) // _NW


def _sc_body(in_hbm, aug_hbm, argsort_hbm, perm_hbm, bufs, asort_v, perm_v,
             rsems, wsems):
    cid = lax.axis_index("c")
    sid = lax.axis_index("s")
    wid = sid * _NC + cid
    base = wid * _TEC_ROWS

    def rd(i):
        b = i % 2
        pltpu.make_async_copy(
            in_hbm.at[pl.ds(base + i * _CH_ROWS, _CH_ROWS)], bufs.at[b],
            rsems.at[b],
        ).start()

    def rdwait(i):
        b = i % 2
        pltpu.make_async_copy(
            in_hbm.at[pl.ds(base + i * _CH_ROWS, _CH_ROWS)], bufs.at[b],
            rsems.at[b],
        ).wait()

    def wr(i):
        b = i % 2
        pltpu.make_async_copy(
            bufs.at[b], aug_hbm.at[pl.ds(base + i * _CH_ROWS, _CH_ROWS)],
            wsems.at[b],
        ).start()

    def wrwait(i):
        b = i % 2
        pltpu.make_async_copy(
            bufs.at[b], aug_hbm.at[pl.ds(base + i * _CH_ROWS, _CH_ROWS)],
            wsems.at[b],
        ).wait()

    rd(0)
    for i in range(_NCH):
        rdwait(i)
        if i + 1 < _NCH:
            if i >= 1:
                wrwait(i - 1)
            rd(i + 1)
        wr(i)
    wrwait(_NCH - 2)
    wrwait(_NCH - 1)

    abase = wid * _ACHUNK
    row_off = lax.rem(abase, N)
    for v in range(_ACHUNK // 16):
        asort_v[pl.ds(v * 16, 16)] = lax.iota(jnp.int32, 16) + (row_off + v * 16)
    pltpu.sync_copy(asort_v, argsort_hbm.at[pl.ds(abase, _ACHUNK)])

    @pl.when(wid == 0)
    def _():
        perm_v[...] = lax.iota(jnp.int32, 16)
        pltpu.sync_copy(perm_v, perm_hbm)


_sc_all = pl.kernel(
    _sc_body,
    out_type=(
        jax.ShapeDtypeStruct((_ROWS, D), jnp.float32),
        jax.ShapeDtypeStruct((NUM_PERM * N,), jnp.int32),
        jax.ShapeDtypeStruct((16,), jnp.int32),
    ),
    mesh=plsc.VectorSubcoreMesh(core_axis_name="c", subcore_axis_name="s"),
    scratch_types=[
        pltpu.VMEM((2, _CH_ROWS, D), jnp.float32),
        pltpu.VMEM((_ACHUNK,), jnp.int32),
        pltpu.VMEM((16,), jnp.int32),
        pltpu.SemaphoreType.DMA((2,)),
        pltpu.SemaphoreType.DMA((2,)),
    ],
)


def kernel(patches):
    aug, argsort_flat, perm16 = _sc_all(patches.reshape(_ROWS, D))
    return (
        aug.reshape(NUM_PERM, C, N, D),
        argsort_flat.reshape(NUM_PERM, N),
        perm16[:NUM_PERM],
    )
